# baseline (speedup 1.0000x reference)
import functools

import jax
import jax.numpy as jnp
from jax import lax
from jax.experimental import pallas as pl
from jax.experimental.pallas import tpu as pltpu
from jax.experimental.pallas import tpu_sc as plsc

B = 4096
L = 200
V = 16384
D = 32
N = B * L
NW = 32
TG = 100
NG = B // NW
NBUF = 4
N4 = N // 4


def _proj_body(emb_ref, w2_ref, b_ref, p_ref):
    p_ref[...] = (
        jnp.dot(emb_ref[...], w2_ref[...], preferred_element_type=jnp.float32)
        + b_ref[...]
    )


def _project(emb_table, w2, b2):
    blk = 2048
    return pl.pallas_call(
        _proj_body,
        grid=(V // blk,),
        in_specs=[
            pl.BlockSpec((blk, D), lambda i: (i, 0)),
            pl.BlockSpec((D, D), lambda i: (0, 0)),
            pl.BlockSpec((1, D), lambda i: (0, 0)),
        ],
        out_specs=pl.BlockSpec((blk, D), lambda i: (i, 0)),
        out_shape=jax.ShapeDtypeStruct((V, D), jnp.float32),
    )(emb_table, w2, b2)


def _make_gather(ng):
    def _gather_body(p_hbm, loc_hbm, out_hbm, idx_v, rows_v, outb_v, gsem, osem):
        cid = lax.axis_index("c")
        sid = lax.axis_index("s")
        wid = sid * 2 + cid

        pltpu.sync_copy(loc_hbm.at[wid], idx_v)

        def gather_copy(g, bslot, h):
            return pltpu.make_async_copy(
                p_hbm.at[idx_v.at[2 * g + h]], rows_v.at[bslot, h], gsem.at[bslot]
            )

        def write_copy(g, bslot):
            word0 = (wid * ng + g) * (50 * 128)
            return pltpu.make_async_copy(
                outb_v.at[bslot], out_hbm.at[pl.ds(word0, 50 * 128)], osem.at[bslot]
            )

        def repack(bslot):
            for h in range(2):
                rv = rows_v.at[bslot, h]
                for sr in range(TG):
                    p = h * TG + sr
                    d0 = (p % 50) * 128 + (p // 50) * 32
                    outb_v[bslot, pl.ds(d0, 16)] = rv[sr, pl.ds(0, 16)]
                    outb_v[bslot, pl.ds(d0 + 16, 16)] = rv[sr, pl.ds(16, 16)]

        def superstep(s, carry):
            for bslot in range(NBUF):
                g = s * NBUF + bslot
                gather_copy(g, bslot, 0).start()
                gather_copy(g, bslot, 1).start()
            for bslot in range(NBUF):
                g = s * NBUF + bslot
                gather_copy(g, bslot, 0).wait()
                gather_copy(g, bslot, 1).wait()

                @pl.when(s > 0)
                def _():
                    write_copy(g - NBUF, bslot).wait()

                repack(bslot)
                write_copy(g, bslot).start()
            return carry

        lax.fori_loop(0, ng // NBUF, superstep, 0, unroll=False)
        for bslot in range(NBUF):
            write_copy(ng - NBUF + bslot, bslot).wait()

    @functools.partial(
        pl.kernel,
        out_type=jax.ShapeDtypeStruct((NW * ng * 50 * 128,), jnp.float32),
        mesh=plsc.VectorSubcoreMesh(core_axis_name="c", subcore_axis_name="s"),
        compiler_params=pltpu.CompilerParams(use_tc_tiling_on_sc=False),
        scratch_types=[
            pltpu.VMEM((2 * ng, TG), jnp.int32),
            pltpu.VMEM((NBUF, 2, TG, D), jnp.float32),
            pltpu.VMEM((NBUF, 50 * 128), jnp.float32),
            pltpu.SemaphoreType.DMA((NBUF,)),
            pltpu.SemaphoreType.DMA((NBUF,)),
        ],
    )
    def _gather(p_hbm, loc_hbm, out_hbm, idx_v, rows_v, outb_v, gsem, osem):
        _gather_body(p_hbm, loc_hbm, out_hbm, idx_v, rows_v, outb_v, gsem, osem)

    return _gather


BB = 16


def _epi_body(lng_ref, lat_ref, w01_ref, g_ref, o_ref):
    lng_t = lng_ref[...].T
    lat_t = lat_ref[...].T
    w01 = w01_ref[...]
    for i in range(BB):
        ll = jnp.concatenate([lng_t[:, i:i + 1], lat_t[:, i:i + 1]], axis=1)
        x = jax.lax.dot_general(
            ll, w01, (((1,), (0,)), ((), ())),
            preferred_element_type=jnp.float32,
        )
        for c in range(4):
            gi = g_ref[pl.ds(i * 50, 50), pl.ds(c * 32, 32)]
            o_ref[i, pl.ds(c * 50, 50), :] = jnp.tanh(
                x[c * 50:(c + 1) * 50, :] + gi
            )


def _epilogue(lng, lat, w01, g4, nb):
    rows = BB * 50
    return pl.pallas_call(
        _epi_body,
        grid=(nb // BB,),
        in_specs=[
            pl.BlockSpec((BB, L), lambda i: (i, 0)),
            pl.BlockSpec((BB, L), lambda i: (i, 0)),
            pl.BlockSpec((2, D), lambda i: (0, 0)),
            pl.BlockSpec((rows, 128), lambda i: (i, 0)),
        ],
        out_specs=pl.BlockSpec((BB, L, D), lambda i: (i, 0, 0)),
        out_shape=jax.ShapeDtypeStruct((nb, L, D), jnp.float32),
    )(lng, lat, w01, g4)


NSPLIT = 2


def kernel(current_longi, current_lati, current_loc, emb_table, W, b):
    w2 = W[2:, :]
    w01 = W[0:2, :]
    b2 = b[None, :]
    p = _project(emb_table, w2, b2)
    bh = B // NSPLIT
    gather = _make_gather(NG // NSPLIT)
    outs = []
    g4s = [
        gather(p, current_loc[k * bh:(k + 1) * bh].reshape(NW, 2 * NG // NSPLIT, TG))
        for k in range(NSPLIT)
    ]
    for k in range(NSPLIT):
        g4 = g4s[k].reshape(bh * L // 4, 128)
        outs.append(
            _epilogue(
                current_longi[k * bh:(k + 1) * bh],
                current_lati[k * bh:(k + 1) * bh],
                w01,
                g4,
                bh,
            )
        )
    return jnp.concatenate(outs, axis=0)


# speedup vs baseline: 1.0600x; 1.0600x over previous
import functools

import jax
import jax.numpy as jnp
from jax import lax
from jax.experimental import pallas as pl
from jax.experimental.pallas import tpu as pltpu
from jax.experimental.pallas import tpu_sc as plsc

B = 4096
L = 200
V = 16384
D = 32
N = B * L
NW = 32
TG = 100
NG = B // NW
NBUF = 4
N4 = N // 4


def _proj_body(emb_ref, w2_ref, b_ref, p_ref):
    p_ref[...] = (
        jnp.dot(emb_ref[...], w2_ref[...], preferred_element_type=jnp.float32)
        + b_ref[...]
    )


def _project(emb_table, w2, b2):
    blk = 2048
    return pl.pallas_call(
        _proj_body,
        grid=(V // blk,),
        in_specs=[
            pl.BlockSpec((blk, D), lambda i: (i, 0)),
            pl.BlockSpec((D, D), lambda i: (0, 0)),
            pl.BlockSpec((1, D), lambda i: (0, 0)),
        ],
        out_specs=pl.BlockSpec((blk, D), lambda i: (i, 0)),
        out_shape=jax.ShapeDtypeStruct((V, D), jnp.float32),
    )(emb_table, w2, b2)


def _make_gather(ng):
    def _gather_body(p_hbm, loc_hbm, out_hbm, idx_v, rows_v, outb_v, gsem, osem):
        cid = lax.axis_index("c")
        sid = lax.axis_index("s")
        wid = sid * 2 + cid

        pltpu.sync_copy(loc_hbm.at[wid], idx_v)

        def gather_copy(g, bslot, h):
            return pltpu.make_async_copy(
                p_hbm.at[idx_v.at[2 * g + h]], rows_v.at[bslot, h], gsem.at[bslot]
            )

        def write_copy(g, bslot):
            word0 = (wid * ng + g) * (50 * 128)
            return pltpu.make_async_copy(
                outb_v.at[bslot], out_hbm.at[pl.ds(word0, 50 * 128)], osem.at[bslot]
            )

        def repack(bslot):
            for h in range(2):
                rv = rows_v.at[bslot, h]
                for sr in range(TG):
                    p = h * TG + sr
                    d0 = (p % 50) * 128 + (p // 50) * 32
                    outb_v[bslot, pl.ds(d0, 16)] = rv[sr, pl.ds(0, 16)]
                    outb_v[bslot, pl.ds(d0 + 16, 16)] = rv[sr, pl.ds(16, 16)]

        def superstep(s, carry):
            for bslot in range(NBUF):
                g = s * NBUF + bslot
                gather_copy(g, bslot, 0).start()
                gather_copy(g, bslot, 1).start()
            for bslot in range(NBUF):
                g = s * NBUF + bslot
                gather_copy(g, bslot, 0).wait()
                gather_copy(g, bslot, 1).wait()

                @pl.when(s > 0)
                def _():
                    write_copy(g - NBUF, bslot).wait()

                repack(bslot)
                write_copy(g, bslot).start()
            return carry

        lax.fori_loop(0, ng // NBUF, superstep, 0, unroll=False)
        for bslot in range(NBUF):
            write_copy(ng - NBUF + bslot, bslot).wait()

    @functools.partial(
        pl.kernel,
        out_type=jax.ShapeDtypeStruct((NW * ng * 50 * 128,), jnp.float32),
        mesh=plsc.VectorSubcoreMesh(core_axis_name="c", subcore_axis_name="s"),
        compiler_params=pltpu.CompilerParams(use_tc_tiling_on_sc=False),
        scratch_types=[
            pltpu.VMEM((2 * ng, TG), jnp.int32),
            pltpu.VMEM((NBUF, 2, TG, D), jnp.float32),
            pltpu.VMEM((NBUF, 50 * 128), jnp.float32),
            pltpu.SemaphoreType.DMA((NBUF,)),
            pltpu.SemaphoreType.DMA((NBUF,)),
        ],
    )
    def _gather(p_hbm, loc_hbm, out_hbm, idx_v, rows_v, outb_v, gsem, osem):
        _gather_body(p_hbm, loc_hbm, out_hbm, idx_v, rows_v, outb_v, gsem, osem)

    return _gather


BB = 16


def _epi_body(lng_ref, lat_ref, w01_ref, g_ref, o_ref):
    lli = jnp.concatenate(
        [lng_ref[...][:, None, :], lat_ref[...][:, None, :]], axis=1
    ).reshape(2 * BB, L).T
    w01 = w01_ref[...]
    for i in range(BB):
        for c in range(4):
            ll_c = lli[c * 50:(c + 1) * 50, 2 * i:2 * i + 2]
            x_c = jax.lax.dot_general(
                ll_c, w01, (((1,), (0,)), ((), ())),
                preferred_element_type=jnp.float32,
            )
            gi = g_ref[pl.ds(i * 50, 50), pl.ds(c * 32, 32)]
            o_ref[i, pl.ds(c * 50, 50), :] = jnp.tanh(x_c + gi)


def _epilogue(lng, lat, w01, g4, nb):
    rows = BB * 50
    return pl.pallas_call(
        _epi_body,
        grid=(nb // BB,),
        in_specs=[
            pl.BlockSpec((BB, L), lambda i: (i, 0)),
            pl.BlockSpec((BB, L), lambda i: (i, 0)),
            pl.BlockSpec((2, D), lambda i: (0, 0)),
            pl.BlockSpec((rows, 128), lambda i: (i, 0)),
        ],
        out_specs=pl.BlockSpec((BB, L, D), lambda i: (i, 0, 0)),
        out_shape=jax.ShapeDtypeStruct((nb, L, D), jnp.float32),
    )(lng, lat, w01, g4)


NSPLIT = 2


def kernel(current_longi, current_lati, current_loc, emb_table, W, b):
    w2 = W[2:, :]
    w01 = W[0:2, :]
    b2 = b[None, :]
    p = _project(emb_table, w2, b2)
    bh = B // NSPLIT
    gather = _make_gather(NG // NSPLIT)
    outs = []
    g4s = [
        gather(p, current_loc[k * bh:(k + 1) * bh].reshape(NW, 2 * NG // NSPLIT, TG))
        for k in range(NSPLIT)
    ]
    for k in range(NSPLIT):
        g4 = g4s[k].reshape(bh * L // 4, 128)
        outs.append(
            _epilogue(
                current_longi[k * bh:(k + 1) * bh],
                current_lati[k * bh:(k + 1) * bh],
                w01,
                g4,
                bh,
            )
        )
    return jnp.concatenate(outs, axis=0)


# speedup vs baseline: 1.0771x; 1.0162x over previous
import functools

import jax
import jax.numpy as jnp
from jax import lax
from jax.experimental import pallas as pl
from jax.experimental.pallas import tpu as pltpu
from jax.experimental.pallas import tpu_sc as plsc

B = 4096
L = 200
V = 16384
D = 32
N = B * L
NW = 32
TG = 100
NG = B // NW
NBUF = 4
N4 = N // 4


def _proj_body(emb_ref, w2_ref, b_ref, p_ref):
    p_ref[...] = (
        jnp.dot(emb_ref[...], w2_ref[...], preferred_element_type=jnp.float32)
        + b_ref[...]
    )


def _project(emb_table, w2, b2):
    blk = 2048
    return pl.pallas_call(
        _proj_body,
        grid=(V // blk,),
        in_specs=[
            pl.BlockSpec((blk, D), lambda i: (i, 0)),
            pl.BlockSpec((D, D), lambda i: (0, 0)),
            pl.BlockSpec((1, D), lambda i: (0, 0)),
        ],
        out_specs=pl.BlockSpec((blk, D), lambda i: (i, 0)),
        out_shape=jax.ShapeDtypeStruct((V, D), jnp.float32),
    )(emb_table, w2, b2)


def _make_gather(ng):
    def _gather_body(p_hbm, loc_hbm, out_hbm, idx_v, rows_v, outb_v, gsem, osem):
        cid = lax.axis_index("c")
        sid = lax.axis_index("s")
        wid = sid * 2 + cid

        pltpu.sync_copy(loc_hbm.at[wid], idx_v)

        def gather_copy(g, bslot, h):
            return pltpu.make_async_copy(
                p_hbm.at[idx_v.at[2 * g + h]], rows_v.at[bslot, h], gsem.at[bslot]
            )

        def write_copy(g, bslot):
            word0 = (wid * ng + g) * (50 * 128)
            return pltpu.make_async_copy(
                outb_v.at[bslot], out_hbm.at[pl.ds(word0, 50 * 128)], osem.at[bslot]
            )

        def repack(bslot):
            for h in range(2):
                rv = rows_v.at[bslot, h]
                for sr in range(TG):
                    p = h * TG + sr
                    d0 = (p % 50) * 128 + (p // 50) * 32
                    outb_v[bslot, pl.ds(d0, 16)] = rv[sr, pl.ds(0, 16)]
                    outb_v[bslot, pl.ds(d0 + 16, 16)] = rv[sr, pl.ds(16, 16)]

        def superstep(s, carry):
            for bslot in range(NBUF):
                g = s * NBUF + bslot
                gather_copy(g, bslot, 0).start()
                gather_copy(g, bslot, 1).start()
            for bslot in range(NBUF):
                g = s * NBUF + bslot
                gather_copy(g, bslot, 0).wait()
                gather_copy(g, bslot, 1).wait()

                @pl.when(s > 0)
                def _():
                    write_copy(g - NBUF, bslot).wait()

                repack(bslot)
                write_copy(g, bslot).start()
            return carry

        lax.fori_loop(0, ng // NBUF, superstep, 0, unroll=False)
        for bslot in range(NBUF):
            write_copy(ng - NBUF + bslot, bslot).wait()

    @functools.partial(
        pl.kernel,
        out_type=jax.ShapeDtypeStruct((NW * ng * 50 * 128,), jnp.float32),
        mesh=plsc.VectorSubcoreMesh(core_axis_name="c", subcore_axis_name="s"),
        compiler_params=pltpu.CompilerParams(use_tc_tiling_on_sc=False),
        scratch_types=[
            pltpu.VMEM((2 * ng, TG), jnp.int32),
            pltpu.VMEM((NBUF, 2, TG, D), jnp.float32),
            pltpu.VMEM((NBUF, 50 * 128), jnp.float32),
            pltpu.SemaphoreType.DMA((NBUF,)),
            pltpu.SemaphoreType.DMA((NBUF,)),
        ],
    )
    def _gather(p_hbm, loc_hbm, out_hbm, idx_v, rows_v, outb_v, gsem, osem):
        _gather_body(p_hbm, loc_hbm, out_hbm, idx_v, rows_v, outb_v, gsem, osem)

    return _gather


BB = 16


def _epi_body(lng_ref, lat_ref, w01_ref, g_ref, o_ref):
    lli = jnp.concatenate(
        [lng_ref[...][:, None, :], lat_ref[...][:, None, :]], axis=1
    ).reshape(2 * BB, L).T
    w01 = w01_ref[...]
    for i in range(BB):
        for c in range(4):
            ll_c = lli[c * 50:(c + 1) * 50, 2 * i:2 * i + 2]
            x_c = jax.lax.dot_general(
                ll_c, w01, (((1,), (0,)), ((), ())),
                preferred_element_type=jnp.float32,
            )
            gi = g_ref[pl.ds(i * 50, 50), pl.ds(c * 32, 32)]
            o_ref[i, pl.ds(c * 50, 50), :] = jnp.tanh(x_c + gi)


def _epilogue(lng, lat, w01, g4, nb):
    rows = BB * 50
    return pl.pallas_call(
        _epi_body,
        grid=(nb // BB,),
        in_specs=[
            pl.BlockSpec((BB, L), lambda i: (i, 0)),
            pl.BlockSpec((BB, L), lambda i: (i, 0)),
            pl.BlockSpec((2, D), lambda i: (0, 0)),
            pl.BlockSpec((rows, 128), lambda i: (i, 0)),
        ],
        out_specs=pl.BlockSpec((BB, L, D), lambda i: (i, 0, 0)),
        out_shape=jax.ShapeDtypeStruct((nb, L, D), jnp.float32),
    )(lng, lat, w01, g4)


NSPLIT = 1


def kernel(current_longi, current_lati, current_loc, emb_table, W, b):
    w2 = W[2:, :]
    w01 = W[0:2, :]
    b2 = b[None, :]
    p = _project(emb_table, w2, b2)
    bh = B // NSPLIT
    gather = _make_gather(NG // NSPLIT)
    outs = []
    g4s = [
        gather(p, current_loc[k * bh:(k + 1) * bh].reshape(NW, 2 * NG // NSPLIT, TG))
        for k in range(NSPLIT)
    ]
    for k in range(NSPLIT):
        g4 = g4s[k].reshape(bh * L // 4, 128)
        outs.append(
            _epilogue(
                current_longi[k * bh:(k + 1) * bh],
                current_lati[k * bh:(k + 1) * bh],
                w01,
                g4,
                bh,
            )
        )
    return jnp.concatenate(outs, axis=0)
